# Initial kernel scaffold; baseline (speedup 1.0000x reference)
import jax
import jax.numpy as jnp
from jax.experimental import pallas as pl


def kernel(p0_enc, v0_enc, p0, v0, a, fluid_mask, Wc0, Wd0, Wc1, Wd1, Wc2, Wd2, Wc3, Wd3, Wc4, Wd4):
    raise NotImplementedError("write your pallas kernel here")


# speedup vs baseline: 1.0893x; 1.0893x over previous
import numpy as np

import jax
import jax.numpy as jnp
from jax.experimental import pallas as pl
from jax.experimental.pallas import tpu as pltpu

K = 8
R = 3
NT = 16
RADIUS = 40.0
N = 512


def _basis_np(k=K):
    th = 2.0 * np.pi * np.arange(k) / k
    return np.stack([np.cos(th), np.sin(th)], axis=-1).astype(np.float32)


def _shift_mats_np(k=K):
    s = np.arange(k)
    tgt = (np.arange(k)[None, :] + s[:, None]) % k
    out = np.zeros((k, k, k), dtype=np.float32)
    for si in range(k):
        for ki in range(k):
            out[si, ki, tgt[si, ki]] = 1.0
    return out


def _wshift(Wc):
    Sall = jnp.asarray(_shift_mats_np())
    Wsh = jnp.einsum('roc,skm->rsckom', Wc, Sall)
    Rr, Ss, Cc, Kk, Oo, Mm = Wsh.shape
    return Wsh.reshape(Rr * Ss, Cc * Kk, Oo * Mm)


def _wdmat(Wd):
    km = (np.arange(K)[None, :] - np.arange(K)[:, None]) % K
    W = Wd[:, :, jnp.asarray(km)]
    W = jnp.transpose(W, (1, 2, 0, 3))
    O = Wd.shape[0]
    C = Wd.shape[1]
    return W.reshape(C * K, O * K)


def _pool(C):
    return jnp.asarray(np.kron(np.eye(C, dtype=np.float32),
                               np.ones((K, K), dtype=np.float32)))


def _net_kernel(pos_row_ref, pos_col_ref, mask_ref, xin_ref,
                wproj_ref, wsh0_ref, wdm0_ref, wsh1_ref, wdm1_ref,
                wsh2_ref, wdm2_ref, wsh3_ref, wdm3_ref, wsh4_ref, wdm4_ref,
                pool64_ref, pool128_ref, out_ref, P_ref):
    f32 = jnp.float32
    px_row = pos_row_ref[0, 0:1, :]
    py_row = pos_row_ref[0, 1:2, :]
    px_col = pos_col_ref[0, :, 0:1]
    py_col = pos_col_ref[0, :, 1:2]
    dx = px_row - px_col
    dy = py_row - py_col
    d = jnp.sqrt(dx * dx + dy * dy + 1e-9)
    rn = jnp.minimum(d * (1.0 / RADIUS), 1.0)
    win = 1.0 - rn * rn
    win = win * win * win
    mask_row = mask_ref[0]
    winm = win * mask_row
    rpos = rn * (R - 1.0)
    ang = jnp.arctan2(dy, dx)
    tb = jnp.floor((ang + np.pi) * (NT / (2.0 * np.pi))).astype(jnp.int32)
    sb = jnp.bitwise_and(tb, K - 1)

    for r in range(R):
        ar = jnp.maximum(1.0 - jnp.abs(rpos - float(r)), 0.0) * winm
        for s in range(K):
            P_ref[r * K + s] = jnp.where(sb == s, ar, 0.0)

    denom = jnp.sum(mask_ref[0])
    inv_denom = 1.0 / (denom + 1e-6)

    def matmul(a, b):
        return jax.lax.dot_general(a, b, (((1,), (0,)), ((), ())),
                                   preferred_element_type=f32)

    def cts(X, wsh_ref, OK):
        acc = jnp.zeros((N, OK), dtype=f32)
        for t in range(R * K):
            Yt = matmul(X, wsh_ref[t])
            acc = acc + matmul(P_ref[t], Yt)
        return acc * inv_denom

    def nonlin(v, pool_ref):
        sq = v * v
        mags = matmul(sq, pool_ref[...]) + 1e-6
        return v * (jnp.maximum(mags - 0.2, 0.0) / mags)

    X0 = matmul(xin_ref[0], wproj_ref[...])

    oc = cts(X0, wsh0_ref, 4 * K)
    od = matmul(X0, wdm0_ref[...])
    prev = jnp.concatenate([oc, od], axis=1)

    X = nonlin(prev, pool64_ref)
    prev = cts(X, wsh1_ref, 8 * K) + matmul(X, wdm1_ref[...]) + prev

    X = nonlin(prev, pool64_ref)
    prev = cts(X, wsh2_ref, 16 * K) + matmul(X, wdm2_ref[...])

    X = nonlin(prev, pool128_ref)
    prev = cts(X, wsh3_ref, 8 * K) + matmul(X, wdm3_ref[...])

    X = nonlin(prev, pool64_ref)
    out_ref[0] = cts(X, wsh4_ref, 3 * K) + matmul(X, wdm4_ref[...])


def kernel(p0_enc, v0_enc, p0, v0, a, fluid_mask,
           Wc0, Wd0, Wc1, Wd1, Wc2, Wd2, Wc3, Wd3, Wc4, Wd4):
    f32 = jnp.float32
    Bm = p0.shape[0]
    dt = 1.0
    v1 = v0 + dt * a
    p1 = p0 + dt * (v0 + v1) / 2.0

    fluid_feats = jnp.concatenate(
        [v1[..., None, :], p1[..., None, :], v0_enc, p0_enc], axis=-2)
    xin = fluid_feats.reshape(Bm, N, 32)

    B8 = jnp.asarray(_basis_np())
    wproj = jnp.kron(jnp.eye(16, dtype=f32), B8.T)

    wsh = [_wshift(Wc0), _wshift(Wc1), _wshift(Wc2), _wshift(Wc3), _wshift(Wc4)]
    wdm = [_wdmat(Wd0), _wdmat(Wd1), _wdmat(Wd2), _wdmat(Wd3), _wdmat(Wd4)]
    pool64 = _pool(8)
    pool128 = _pool(16)

    pos_row = jnp.transpose(p1, (0, 2, 1))
    maskr = fluid_mask[:, None, :]

    def rep(arr):
        nd = arr.ndim
        return pl.BlockSpec(arr.shape, lambda b: (0,) * nd)

    in_arrays = [pos_row, p1, maskr, xin,
                 wproj, wsh[0], wdm[0], wsh[1], wdm[1],
                 wsh[2], wdm[2], wsh[3], wdm[3], wsh[4], wdm[4],
                 pool64, pool128]
    in_specs = [
        pl.BlockSpec((1, 2, N), lambda b: (b, 0, 0)),
        pl.BlockSpec((1, N, 2), lambda b: (b, 0, 0)),
        pl.BlockSpec((1, 1, N), lambda b: (b, 0, 0)),
        pl.BlockSpec((1, N, 32), lambda b: (b, 0, 0)),
    ] + [rep(w) for w in in_arrays[4:]]

    out_reg = pl.pallas_call(
        _net_kernel,
        grid=(Bm,),
        in_specs=in_specs,
        out_specs=pl.BlockSpec((1, N, 3 * K), lambda b: (b, 0, 0)),
        out_shape=jax.ShapeDtypeStruct((Bm, N, 3 * K), f32),
        scratch_shapes=[pltpu.VMEM((R * K, N, N), f32)],
    )(*in_arrays)

    out3 = out_reg.reshape(Bm, N, 3, K)
    pos_correction = (1.0 / 128.0) * (2.0 / K) * jnp.einsum('bnck,kd->bncd', out3, B8)
    p_corrected = p1 + pos_correction[..., 0, :]
    v_corrected = (p_corrected - p0) / dt
    m_matrix = pos_correction[..., 1:, :]
    return p_corrected, v_corrected, m_matrix, (v0_enc, p0_enc)


# speedup vs baseline: 1.3528x; 1.2419x over previous
import numpy as np

import jax
import jax.numpy as jnp
from jax.experimental import pallas as pl
from jax.experimental.pallas import tpu as pltpu

K = 8
R = 3
NT = 16
RADIUS = 40.0
N = 512


def _basis_np(k=K):
    th = 2.0 * np.pi * np.arange(k) / k
    return np.stack([np.cos(th), np.sin(th)], axis=-1).astype(np.float32)


def _shift_mats_np(k=K):
    s = np.arange(k)
    tgt = (np.arange(k)[None, :] + s[:, None]) % k
    out = np.zeros((k, k, k), dtype=np.float32)
    for si in range(k):
        for ki in range(k):
            out[si, ki, tgt[si, ki]] = 1.0
    return out


def _wshift(Wc):
    Sall = jnp.asarray(_shift_mats_np())
    Wsh = jnp.einsum('roc,skm->rsckom', Wc, Sall)
    Rr, Ss, Cc, Kk, Oo, Mm = Wsh.shape
    Wsh = Wsh.reshape(Rr * Ss, Cc * Kk, Oo * Mm)
    return jnp.transpose(Wsh, (1, 0, 2)).reshape(Cc * Kk, Rr * Ss * Oo * Mm)


def _wdmat(Wd):
    km = (np.arange(K)[None, :] - np.arange(K)[:, None]) % K
    W = Wd[:, :, jnp.asarray(km)]
    W = jnp.transpose(W, (1, 2, 0, 3))
    O = Wd.shape[0]
    C = Wd.shape[1]
    return W.reshape(C * K, O * K)


def _pool(C):
    return jnp.asarray(np.kron(np.eye(C, dtype=np.float32),
                               np.ones((K, K), dtype=np.float32)))


def _net_kernel(pos_row_ref, pos_col_ref, mask_ref, xin_ref,
                wproj_ref, wsh0_ref, wdm0_ref, wsh1_ref, wdm1_ref,
                wsh2_ref, wdm2_ref, wsh3_ref, wdm3_ref, wsh4_ref, wdm4_ref,
                pool64_ref, pool128_ref, out_ref, P_ref):
    f32 = jnp.float32
    px_row = pos_row_ref[0, 0:1, :]
    py_row = pos_row_ref[0, 1:2, :]
    px_col = pos_col_ref[0, :, 0:1]
    py_col = pos_col_ref[0, :, 1:2]
    dx = px_row - px_col
    dy = py_row - py_col
    d = jnp.sqrt(dx * dx + dy * dy + 1e-9)
    rn = jnp.minimum(d * (1.0 / RADIUS), 1.0)
    win = 1.0 - rn * rn
    win = win * win * win
    mask_row = mask_ref[0]
    winm = win * mask_row
    rpos = rn * (R - 1.0)
    ang = jnp.arctan2(dy, dx)
    tb = jnp.floor((ang + np.pi) * (NT / (2.0 * np.pi))).astype(jnp.int32)
    sb = jnp.bitwise_and(tb, K - 1)

    for r in range(R):
        ar = jnp.maximum(1.0 - jnp.abs(rpos - float(r)), 0.0) * winm
        for s in range(K):
            P_ref[r * K + s] = jnp.where(sb == s, ar, 0.0).astype(jnp.bfloat16)

    denom = jnp.sum(mask_ref[0])
    inv_denom = 1.0 / (denom + 1e-6)

    def matmul(a, b):
        return jax.lax.dot_general(a, b, (((1,), (0,)), ((), ())),
                                   preferred_element_type=f32)

    def cts(X, wsh_ref, OK):
        Yall = matmul(X, wsh_ref[...]).astype(jnp.bfloat16)
        acc = jnp.zeros((N, OK), dtype=f32)
        for t in range(R * K):
            acc = acc + matmul(P_ref[t], Yall[:, t * OK:(t + 1) * OK])
        return acc * inv_denom

    def nonlin(v, pool_ref):
        sq = v * v
        mags = matmul(sq, pool_ref[...]) + 1e-6
        return v * (jnp.maximum(mags - 0.2, 0.0) / mags)

    X0 = matmul(xin_ref[0], wproj_ref[...])

    oc = cts(X0, wsh0_ref, 4 * K)
    od = matmul(X0, wdm0_ref[...])
    prev = jnp.concatenate([oc, od], axis=1)

    X = nonlin(prev, pool64_ref)
    prev = cts(X, wsh1_ref, 8 * K) + matmul(X, wdm1_ref[...]) + prev

    X = nonlin(prev, pool64_ref)
    prev = cts(X, wsh2_ref, 16 * K) + matmul(X, wdm2_ref[...])

    X = nonlin(prev, pool128_ref)
    prev = cts(X, wsh3_ref, 8 * K) + matmul(X, wdm3_ref[...])

    X = nonlin(prev, pool64_ref)
    out_ref[0] = cts(X, wsh4_ref, 3 * K) + matmul(X, wdm4_ref[...])


def kernel(p0_enc, v0_enc, p0, v0, a, fluid_mask,
           Wc0, Wd0, Wc1, Wd1, Wc2, Wd2, Wc3, Wd3, Wc4, Wd4):
    f32 = jnp.float32
    Bm = p0.shape[0]
    dt = 1.0
    v1 = v0 + dt * a
    p1 = p0 + dt * (v0 + v1) / 2.0

    fluid_feats = jnp.concatenate(
        [v1[..., None, :], p1[..., None, :], v0_enc, p0_enc], axis=-2)
    xin = fluid_feats.reshape(Bm, N, 32)

    B8 = jnp.asarray(_basis_np())
    wproj = jnp.kron(jnp.eye(16, dtype=f32), B8.T)

    wsh = [_wshift(Wc0), _wshift(Wc1), _wshift(Wc2), _wshift(Wc3), _wshift(Wc4)]
    wdm = [_wdmat(Wd0), _wdmat(Wd1), _wdmat(Wd2), _wdmat(Wd3), _wdmat(Wd4)]
    pool64 = _pool(8)
    pool128 = _pool(16)

    pos_row = jnp.transpose(p1, (0, 2, 1))
    maskr = fluid_mask[:, None, :]

    def rep(arr):
        nd = arr.ndim
        return pl.BlockSpec(arr.shape, lambda b: (0,) * nd)

    in_arrays = [pos_row, p1, maskr, xin,
                 wproj, wsh[0], wdm[0], wsh[1], wdm[1],
                 wsh[2], wdm[2], wsh[3], wdm[3], wsh[4], wdm[4],
                 pool64, pool128]
    in_specs = [
        pl.BlockSpec((1, 2, N), lambda b: (b, 0, 0)),
        pl.BlockSpec((1, N, 2), lambda b: (b, 0, 0)),
        pl.BlockSpec((1, 1, N), lambda b: (b, 0, 0)),
        pl.BlockSpec((1, N, 32), lambda b: (b, 0, 0)),
    ] + [rep(w) for w in in_arrays[4:]]

    out_reg = pl.pallas_call(
        _net_kernel,
        grid=(Bm,),
        in_specs=in_specs,
        out_specs=pl.BlockSpec((1, N, 3 * K), lambda b: (b, 0, 0)),
        out_shape=jax.ShapeDtypeStruct((Bm, N, 3 * K), f32),
        scratch_shapes=[pltpu.VMEM((R * K, N, N), jnp.bfloat16)],
    )(*in_arrays)

    out3 = out_reg.reshape(Bm, N, 3, K)
    pos_correction = (1.0 / 128.0) * (2.0 / K) * jnp.einsum('bnck,kd->bncd', out3, B8)
    p_corrected = p1 + pos_correction[..., 0, :]
    v_corrected = (p_corrected - p0) / dt
    m_matrix = pos_correction[..., 1:, :]
    return p_corrected, v_corrected, m_matrix, (v0_enc, p0_enc)
